# Initial kernel scaffold; baseline (speedup 1.0000x reference)
import jax
import jax.numpy as jnp
from jax.experimental import pallas as pl


def kernel(x, edge_index, W1, b1, Wmu, bmu, Wlv, blv, eps):
    raise NotImplementedError("write your pallas kernel here")


# speedup vs baseline: 3.5652x; 3.5652x over previous
import functools

import jax
import jax.numpy as jnp
from jax.experimental import pallas as pl
from jax.experimental.pallas import tpu as pltpu


def _mm_scale_body(x_ref, w_ref, dinv_ref, o_ref):
    acc = jnp.dot(x_ref[...], w_ref[...], preferred_element_type=jnp.float32)
    o_ref[...] = acc * dinv_ref[...]


def _mm_scale(x, w, dinv_col, block_rows=1000):
    n, d = x.shape
    h = w.shape[1]
    grid = (n // block_rows,)
    return pl.pallas_call(
        _mm_scale_body,
        grid=grid,
        in_specs=[
            pl.BlockSpec((block_rows, d), lambda i: (i, 0)),
            pl.BlockSpec((d, h), lambda i: (0, 0)),
            pl.BlockSpec((block_rows, 1), lambda i: (i, 0)),
        ],
        out_specs=pl.BlockSpec((block_rows, h), lambda i: (i, 0)),
        out_shape=jax.ShapeDtypeStruct((n, h), jnp.float32),
    )(x, w, dinv_col)


def _layer1_body(agg_ref, hn_ref, dinv_ref, b_ref, w_ref, o_ref):
    h = jnp.maximum(dinv_ref[...] * (agg_ref[...] + hn_ref[...]) + b_ref[...], 0.0)
    acc = jnp.dot(h, w_ref[...], preferred_element_type=jnp.float32)
    o_ref[...] = acc * dinv_ref[...]


def _layer1(agg, hn, dinv_col, b_row, wml, block_rows=1000):
    n, d = agg.shape
    h2 = wml.shape[1]
    grid = (n // block_rows,)
    return pl.pallas_call(
        _layer1_body,
        grid=grid,
        in_specs=[
            pl.BlockSpec((block_rows, d), lambda i: (i, 0)),
            pl.BlockSpec((block_rows, d), lambda i: (i, 0)),
            pl.BlockSpec((block_rows, 1), lambda i: (i, 0)),
            pl.BlockSpec((1, d), lambda i: (0, 0)),
            pl.BlockSpec((d, h2), lambda i: (0, 0)),
        ],
        out_specs=pl.BlockSpec((block_rows, h2), lambda i: (i, 0)),
        out_shape=jax.ShapeDtypeStruct((n, h2), jnp.float32),
    )(agg, hn, dinv_col, b_row, wml)


def _head_body(agg_ref, pn_ref, dinv_ref, b_ref, eps_ref, mu_ref, lv_ref, z_ref):
    mulv = dinv_ref[...] * (agg_ref[...] + pn_ref[...]) + b_ref[...]
    l = mu_ref.shape[1]
    mu = mulv[:, :l]
    lv = mulv[:, l:]
    mu_ref[...] = mu
    lv_ref[...] = lv
    z_ref[...] = mu + eps_ref[...] * jnp.exp(0.5 * lv)


def _head(agg, pn, dinv_col, b_row, eps, block_rows=1000):
    n, h2 = agg.shape
    l = h2 // 2
    grid = (n // block_rows,)
    out_shape = [jax.ShapeDtypeStruct((n, l), jnp.float32)] * 3
    return pl.pallas_call(
        _head_body,
        grid=grid,
        in_specs=[
            pl.BlockSpec((block_rows, h2), lambda i: (i, 0)),
            pl.BlockSpec((block_rows, h2), lambda i: (i, 0)),
            pl.BlockSpec((block_rows, 1), lambda i: (i, 0)),
            pl.BlockSpec((1, h2), lambda i: (0, 0)),
            pl.BlockSpec((block_rows, l), lambda i: (i, 0)),
        ],
        out_specs=[pl.BlockSpec((block_rows, l), lambda i: (i, 0))] * 3,
        out_shape=out_shape,
    )(agg, pn, dinv_col, b_row, eps)


def _decode_body(z_ref, zt_ref, o_ref):
    acc = jnp.dot(z_ref[...], zt_ref[...], preferred_element_type=jnp.float32)
    o_ref[...] = jax.nn.sigmoid(acc)


def _decode(z, block_rows=512, block_cols=2048):
    n, l = z.shape
    zt = z.T
    grid = (pl.cdiv(n, block_rows), pl.cdiv(n, block_cols))
    return pl.pallas_call(
        _decode_body,
        grid=grid,
        in_specs=[
            pl.BlockSpec((block_rows, l), lambda i, j: (i, 0)),
            pl.BlockSpec((l, block_cols), lambda i, j: (0, j)),
        ],
        out_specs=pl.BlockSpec((block_rows, block_cols), lambda i, j: (i, j)),
        out_shape=jax.ShapeDtypeStruct((n, n), jnp.float32),
    )(z, zt)


def kernel(x, edge_index, W1, b1, Wmu, bmu, Wlv, blv, eps):
    n = x.shape[0]
    src = edge_index[0]
    dst = edge_index[1]
    f32 = jnp.float32

    deg = jnp.ones((n,), f32).at[dst].add(1.0)
    dinv = jax.lax.rsqrt(deg)
    dinv_col = dinv[:, None]

    hn1 = _mm_scale(x, W1, dinv_col)
    agg1 = jnp.zeros_like(hn1).at[dst].add(hn1[src])

    wml = jnp.concatenate([Wmu, Wlv], axis=1)
    bml = jnp.concatenate([bmu, blv])[None, :]
    pn = _layer1(agg1, hn1, dinv_col, b1[None, :], wml)
    agg2 = jnp.zeros_like(pn).at[dst].add(pn[src])

    mu, logvar, z = _head(agg2, pn, dinv_col, bml, eps)
    recon = _decode(z)
    return (recon, mu, logvar)


# speedup vs baseline: 13.4893x; 3.7836x over previous
import functools

import jax
import jax.numpy as jnp
from jax import lax
from jax.experimental import pallas as pl
from jax.experimental.pallas import tpu as pltpu
from jax.experimental.pallas import tpu_sc as plsc

N_PAD = 10240
HALF = N_PAD // 2
TRASH = 1024
R = HALF + TRASH
CHUNK = 128
NBUF = 2
NSUB = 16


def _remap(dst_v, j, chalf):
    for g in range(CHUNK // 16):
        v = dst_v[j, pl.ds(g * 16, 16)]
        rel = v - chalf
        inb = (rel >= 0) & (rel < HALF)
        tr = HALF + (v & (TRASH - 1))
        dst_v[j, pl.ds(g * 16, 16)] = jnp.where(inb, rel, tr)


def _sc_agg_call(hn_pad, src3, dst3):
    n_pad, w = hn_pad.shape
    n_chunks = src3.shape[1]
    rpt = R // NSUB
    nzc = rpt // CHUNK
    assert rpt % CHUNK == 0 and n_chunks % NBUF == 0
    mesh = plsc.VectorSubcoreMesh(core_axis_name="c", subcore_axis_name="s")

    @functools.partial(
        pl.kernel,
        out_type=pltpu.HBM((2, R, w), jnp.float32),
        mesh=mesh,
        scratch_types=[
            pltpu.VMEM((n_chunks, CHUNK), jnp.int32),
            pltpu.VMEM((n_chunks, CHUNK), jnp.int32),
            *[pltpu.VMEM((CHUNK, w), jnp.float32) for _ in range(NBUF)],
            pltpu.VMEM_SHARED((R, w), jnp.float32),
            *[pltpu.SemaphoreType.DMA for _ in range(NBUF)],
        ],
    )
    def k(hn_hbm, src_hbm, dst_hbm, zpat_hbm, out_hbm, src_v, dst_v, *rest):
        bufs = rest[:NBUF]
        table = rest[NBUF]
        sems = rest[NBUF + 1:]
        c = lax.axis_index("c")
        s = lax.axis_index("s")
        r0 = s * rpt
        chalf = c * HALF

        pltpu.sync_copy(zpat_hbm, bufs[0])
        for q in range(nzc):
            pltpu.sync_copy(bufs[0], table.at[pl.ds(r0 + q * CHUNK, CHUNK)])

        pltpu.sync_copy(src_hbm.at[s], src_v)
        pltpu.sync_copy(dst_hbm.at[s], dst_v)
        plsc.subcore_barrier()

        def group(t, carry):
            base = t * NBUF
            descs = [
                pltpu.async_copy(hn_hbm.at[src_v.at[base + b]], bufs[b], sems[b])
                for b in range(NBUF)
            ]
            for b in range(NBUF):
                _remap(dst_v, base + b, chalf)
                descs[b].wait()
                pltpu.sync_copy(bufs[b], table.at[dst_v.at[base + b]], add=True)
            return carry
        lax.fori_loop(0, n_chunks // NBUF, group, 0)
        plsc.subcore_barrier()

        pltpu.sync_copy(
            table.at[pl.ds(r0, rpt)],
            out_hbm.at[c, pl.ds(r0, rpt)],
        )

    zpat = jnp.zeros((CHUNK, w), jnp.float32)
    return k(hn_pad, src3, dst3, zpat)


def _sc_deg_call(dst3):
    w = 16
    n_chunks = dst3.shape[1]
    rpt = R // NSUB
    nzc = rpt // CHUNK
    mesh = plsc.VectorSubcoreMesh(core_axis_name="c", subcore_axis_name="s")

    @functools.partial(
        pl.kernel,
        out_type=pltpu.HBM((2, R, w), jnp.float32),
        mesh=mesh,
        scratch_types=[
            pltpu.VMEM((n_chunks, CHUNK), jnp.int32),
            pltpu.VMEM((CHUNK, w), jnp.float32),
            pltpu.VMEM((CHUNK, w), jnp.float32),
            pltpu.VMEM_SHARED((R, w), jnp.float32),
        ],
    )
    def k(dst_hbm, zpat_hbm, opat_hbm, out_hbm, dst_v, zbuf, obuf, table):
        c = lax.axis_index("c")
        s = lax.axis_index("s")
        r0 = s * rpt
        chalf = c * HALF

        pltpu.sync_copy(zpat_hbm, zbuf)
        pltpu.sync_copy(opat_hbm, obuf)
        for q in range(nzc):
            pltpu.sync_copy(zbuf, table.at[pl.ds(r0 + q * CHUNK, CHUNK)])
        pltpu.sync_copy(dst_hbm.at[s], dst_v)
        plsc.subcore_barrier()

        def chunk(j, carry):
            _remap(dst_v, j, chalf)
            pltpu.sync_copy(obuf, table.at[dst_v.at[j]], add=True)
            return carry
        lax.fori_loop(0, n_chunks, chunk, 0)
        plsc.subcore_barrier()

        pltpu.sync_copy(
            table.at[pl.ds(r0, rpt)],
            out_hbm.at[c, pl.ds(r0, rpt)],
        )

    zpat = jnp.zeros((CHUNK, w), jnp.float32)
    opat = jnp.tile(jax.nn.one_hot(0, w, dtype=jnp.float32), (CHUNK, 1))
    return k(dst3, zpat, opat)


def _mm_scale_body(x_ref, w_ref, dinv_ref, o_ref):
    acc = jnp.dot(x_ref[...], w_ref[...], preferred_element_type=jnp.float32)
    o_ref[...] = acc * dinv_ref[...]


def _mm_scale(x, w, dinv_col, n_out, block_rows=1024):
    n, d = x.shape
    h = w.shape[1]
    grid = (n_out // block_rows,)
    return pl.pallas_call(
        _mm_scale_body,
        grid=grid,
        in_specs=[
            pl.BlockSpec((block_rows, d), lambda i: (i, 0)),
            pl.BlockSpec((d, h), lambda i: (0, 0)),
            pl.BlockSpec((block_rows, 1), lambda i: (i, 0)),
        ],
        out_specs=pl.BlockSpec((block_rows, h), lambda i: (i, 0)),
        out_shape=jax.ShapeDtypeStruct((n_out, h), jnp.float32),
    )(x, w, dinv_col)


def _layer1_body(agg_ref, hn_ref, dinv_ref, b_ref, o_ref):
    h = jnp.maximum(
        dinv_ref[...] * (agg_ref[...] + hn_ref[...]) + b_ref[...], 0.0)
    o_ref[...] = h * dinv_ref[...]


def _layer1(agg, hn, dinv_col, b_row, block_rows=1024):
    n, d = hn.shape
    grid = (n // block_rows,)
    return pl.pallas_call(
        _layer1_body,
        grid=grid,
        in_specs=[
            pl.BlockSpec((block_rows, d), lambda i: (i, 0)),
            pl.BlockSpec((block_rows, d), lambda i: (i, 0)),
            pl.BlockSpec((block_rows, 1), lambda i: (i, 0)),
            pl.BlockSpec((1, d), lambda i: (0, 0)),
        ],
        out_specs=pl.BlockSpec((block_rows, d), lambda i: (i, 0)),
        out_shape=jax.ShapeDtypeStruct((n, d), jnp.float32),
    )(agg, hn, dinv_col, b_row)


def _head_body(agg_ref, hn_ref, dinv_ref, w_ref, b_ref, eps_ref,
               mu_ref, lv_ref, z_ref):
    g = dinv_ref[...] * (agg_ref[...] + hn_ref[...])
    mulv = jnp.dot(g, w_ref[...], preferred_element_type=jnp.float32) \
        + b_ref[...]
    l = mu_ref.shape[1]
    mu = mulv[:, :l]
    lv = mulv[:, l:]
    mu_ref[...] = mu
    lv_ref[...] = lv
    z_ref[...] = mu + eps_ref[...] * jnp.exp(0.5 * lv)


def _head(agg, hn, dinv_col, wml, bml, eps, block_rows=1000):
    n, l = eps.shape
    d = hn.shape[1]
    h2 = wml.shape[1]
    grid = (n // block_rows,)
    out_shape = [jax.ShapeDtypeStruct((n, l), jnp.float32)] * 3
    return pl.pallas_call(
        _head_body,
        grid=grid,
        in_specs=[
            pl.BlockSpec((block_rows, d), lambda i: (i, 0)),
            pl.BlockSpec((block_rows, d), lambda i: (i, 0)),
            pl.BlockSpec((block_rows, 1), lambda i: (i, 0)),
            pl.BlockSpec((d, h2), lambda i: (0, 0)),
            pl.BlockSpec((1, h2), lambda i: (0, 0)),
            pl.BlockSpec((block_rows, l), lambda i: (i, 0)),
        ],
        out_specs=[pl.BlockSpec((block_rows, l), lambda i: (i, 0))] * 3,
        out_shape=out_shape,
    )(agg, hn, dinv_col, wml, bml, eps)


def _decode_body(z_ref, zt_ref, o_ref):
    acc = jnp.dot(z_ref[...], zt_ref[...], preferred_element_type=jnp.float32)
    o_ref[...] = jax.nn.sigmoid(acc)


def _decode(z, block_rows=512, block_cols=2048):
    n, l = z.shape
    zt = z.T
    grid = (pl.cdiv(n, block_rows), pl.cdiv(n, block_cols))
    return pl.pallas_call(
        _decode_body,
        grid=grid,
        in_specs=[
            pl.BlockSpec((block_rows, l), lambda i, j: (i, 0)),
            pl.BlockSpec((l, block_cols), lambda i, j: (0, j)),
        ],
        out_specs=pl.BlockSpec((block_rows, block_cols), lambda i, j: (i, j)),
        out_shape=jax.ShapeDtypeStruct((n, n), jnp.float32),
    )(z, zt)


def kernel(x, edge_index, W1, b1, Wmu, bmu, Wlv, blv, eps):
    n = x.shape[0]
    e = edge_index.shape[1]

    grp = NSUB * CHUNK * NBUF
    ept = ((e + grp - 1) // grp) * CHUNK * NBUF
    e_pad = ept * NSUB
    n_spare = N_PAD - n

    src = edge_index[0]
    dst = edge_index[1]
    pad_idx = (n + jnp.arange(e_pad - e, dtype=jnp.int32) % n_spare)
    src3 = jnp.concatenate([src, pad_idx]).reshape(NSUB, ept // CHUNK, CHUNK)
    dst3 = jnp.concatenate([dst, pad_idx]).reshape(NSUB, ept // CHUNK, CHUNK)

    deg_t = _sc_deg_call(dst3)
    deg = 1.0 + deg_t[:, :HALF, 0].reshape(N_PAD)
    dinv_col = jax.lax.rsqrt(deg)[:, None]

    hn1 = _mm_scale(x, W1, dinv_col, N_PAD)
    agg1 = _sc_agg_call(hn1, src3, dst3)[:, :HALF].reshape(N_PAD, -1)

    hn2 = _layer1(agg1, hn1, dinv_col, b1[None, :])
    agg2 = _sc_agg_call(hn2, src3, dst3)[:, :HALF].reshape(N_PAD, -1)

    wml = jnp.concatenate([Wmu, Wlv], axis=1)
    bml = jnp.concatenate([bmu, blv])[None, :]
    mu, logvar, z = _head(agg2[:n], hn2[:n], dinv_col[:n], wml, bml, eps)
    recon = _decode(z)
    return (recon, mu, logvar)


# speedup vs baseline: 16.6701x; 1.2358x over previous
import functools

import jax
import jax.numpy as jnp
from jax import lax
from jax.experimental import pallas as pl
from jax.experimental.pallas import tpu as pltpu
from jax.experimental.pallas import tpu_sc as plsc

N_PAD = 10240
HALF = N_PAD // 2
TRASH = 1024
R = HALF + TRASH
CHUNK = 128
NBUF = 2
NSUB = 16


def _remap(dst_v, j, chalf):
    for g in range(CHUNK // 16):
        v = dst_v[j, pl.ds(g * 16, 16)]
        rel = v - chalf
        inb = (rel >= 0) & (rel < HALF)
        tr = HALF + (v & (TRASH - 1))
        dst_v[j, pl.ds(g * 16, 16)] = jnp.where(inb, rel, tr)


def _sc_agg_call(hn_pad, src3, dst3):
    n_pad, w = hn_pad.shape
    n_chunks = src3.shape[1]
    rpt = R // NSUB
    nzc = rpt // CHUNK
    assert rpt % CHUNK == 0 and n_chunks % NBUF == 0
    mesh = plsc.VectorSubcoreMesh(core_axis_name="c", subcore_axis_name="s")

    @functools.partial(
        pl.kernel,
        out_type=pltpu.HBM((2, R, w), jnp.float32),
        mesh=mesh,
        scratch_types=[
            pltpu.VMEM((n_chunks, CHUNK), jnp.int32),
            pltpu.VMEM((n_chunks, CHUNK), jnp.int32),
            *[pltpu.VMEM((CHUNK, w), jnp.float32) for _ in range(NBUF)],
            pltpu.VMEM_SHARED((R, w), jnp.float32),
            *[pltpu.SemaphoreType.DMA for _ in range(NBUF)],
        ],
    )
    def k(hn_hbm, src_hbm, dst_hbm, zpat_hbm, out_hbm, src_v, dst_v, *rest):
        bufs = rest[:NBUF]
        table = rest[NBUF]
        sems = rest[NBUF + 1:]
        c = lax.axis_index("c")
        s = lax.axis_index("s")
        r0 = s * rpt
        chalf = c * HALF

        pltpu.sync_copy(zpat_hbm, bufs[0])
        for q in range(nzc):
            pltpu.sync_copy(bufs[0], table.at[pl.ds(r0 + q * CHUNK, CHUNK)])

        pltpu.sync_copy(src_hbm.at[s], src_v)
        pltpu.sync_copy(dst_hbm.at[s], dst_v)
        plsc.subcore_barrier()

        for b in range(NBUF):
            pltpu.async_copy(hn_hbm.at[src_v.at[b]], bufs[b], sems[b])

        def group(t, carry):
            base = t * NBUF
            for b in range(NBUF):
                _remap(dst_v, base + b, chalf)
                pltpu.make_async_copy(
                    hn_hbm.at[src_v.at[base + b]], bufs[b], sems[b]).wait()
                pltpu.sync_copy(bufs[b], table.at[dst_v.at[base + b]],
                                add=True)
                pltpu.async_copy(
                    hn_hbm.at[src_v.at[base + NBUF + b]], bufs[b], sems[b])
            return carry
        lax.fori_loop(0, n_chunks // NBUF - 1, group, 0)

        base = n_chunks - NBUF
        for b in range(NBUF):
            _remap(dst_v, base + b, chalf)
            pltpu.make_async_copy(
                hn_hbm.at[src_v.at[base + b]], bufs[b], sems[b]).wait()
            pltpu.sync_copy(bufs[b], table.at[dst_v.at[base + b]], add=True)
        plsc.subcore_barrier()

        pltpu.sync_copy(
            table.at[pl.ds(r0, rpt)],
            out_hbm.at[c, pl.ds(r0, rpt)],
        )

    zpat = jnp.zeros((CHUNK, w), jnp.float32)
    return k(hn_pad, src3, dst3, zpat)


def _sc_deg_call(dst3):
    w = 16
    n_chunks = dst3.shape[1]
    rpt = R // NSUB
    nzc = rpt // CHUNK
    mesh = plsc.VectorSubcoreMesh(core_axis_name="c", subcore_axis_name="s")

    @functools.partial(
        pl.kernel,
        out_type=pltpu.HBM((2, R, w), jnp.float32),
        mesh=mesh,
        scratch_types=[
            pltpu.VMEM((n_chunks, CHUNK), jnp.int32),
            pltpu.VMEM((CHUNK, w), jnp.float32),
            pltpu.VMEM((CHUNK, w), jnp.float32),
            pltpu.VMEM_SHARED((R, w), jnp.float32),
        ],
    )
    def k(dst_hbm, zpat_hbm, opat_hbm, out_hbm, dst_v, zbuf, obuf, table):
        c = lax.axis_index("c")
        s = lax.axis_index("s")
        r0 = s * rpt
        chalf = c * HALF

        pltpu.sync_copy(zpat_hbm, zbuf)
        pltpu.sync_copy(opat_hbm, obuf)
        for q in range(nzc):
            pltpu.sync_copy(zbuf, table.at[pl.ds(r0 + q * CHUNK, CHUNK)])
        pltpu.sync_copy(dst_hbm.at[s], dst_v)
        plsc.subcore_barrier()

        def chunk(j, carry):
            _remap(dst_v, j, chalf)
            pltpu.sync_copy(obuf, table.at[dst_v.at[j]], add=True)
            return carry
        lax.fori_loop(0, n_chunks, chunk, 0)
        plsc.subcore_barrier()

        pltpu.sync_copy(
            table.at[pl.ds(r0, rpt)],
            out_hbm.at[c, pl.ds(r0, rpt)],
        )

    zpat = jnp.zeros((CHUNK, w), jnp.float32)
    opat = jnp.tile(jax.nn.one_hot(0, w, dtype=jnp.float32), (CHUNK, 1))
    return k(dst3, zpat, opat)


def _mm_body(x_ref, w_ref, o_ref):
    o_ref[...] = jnp.dot(x_ref[...], w_ref[...],
                         preferred_element_type=jnp.float32)


def _mm(x, w, n_out, block_rows=1024):
    n, d = x.shape
    h = w.shape[1]
    grid = (n_out // block_rows,)
    return pl.pallas_call(
        _mm_body,
        grid=grid,
        in_specs=[
            pl.BlockSpec((block_rows, d), lambda i: (i, 0)),
            pl.BlockSpec((d, h), lambda i: (0, 0)),
        ],
        out_specs=pl.BlockSpec((block_rows, h), lambda i: (i, 0)),
        out_shape=jax.ShapeDtypeStruct((n_out, h), jnp.float32),
    )(x, w)


def _scale_body(x_ref, dinv_ref, o_ref):
    o_ref[...] = x_ref[...] * dinv_ref[...]


def _scale(x, dinv_col, block_rows=1024):
    n, d = x.shape
    grid = (n // block_rows,)
    return pl.pallas_call(
        _scale_body,
        grid=grid,
        in_specs=[
            pl.BlockSpec((block_rows, d), lambda i: (i, 0)),
            pl.BlockSpec((block_rows, 1), lambda i: (i, 0)),
        ],
        out_specs=pl.BlockSpec((block_rows, d), lambda i: (i, 0)),
        out_shape=jax.ShapeDtypeStruct((n, d), jnp.float32),
    )(x, dinv_col)


def _layer1_body(agg_ref, hn_ref, dinv_ref, b_ref, o_ref):
    h = jnp.maximum(
        dinv_ref[...] * (agg_ref[...] + hn_ref[...]) + b_ref[...], 0.0)
    o_ref[...] = h * dinv_ref[...]


def _layer1(agg, hn, dinv_col, b_row, block_rows=1024):
    n, d = hn.shape
    grid = (n // block_rows,)
    return pl.pallas_call(
        _layer1_body,
        grid=grid,
        in_specs=[
            pl.BlockSpec((block_rows, d), lambda i: (i, 0)),
            pl.BlockSpec((block_rows, d), lambda i: (i, 0)),
            pl.BlockSpec((block_rows, 1), lambda i: (i, 0)),
            pl.BlockSpec((1, d), lambda i: (0, 0)),
        ],
        out_specs=pl.BlockSpec((block_rows, d), lambda i: (i, 0)),
        out_shape=jax.ShapeDtypeStruct((n, d), jnp.float32),
    )(agg, hn, dinv_col, b_row)


def _head_body(agg_ref, hn_ref, dinv_ref, w_ref, b_ref, eps_ref,
               mu_ref, lv_ref, z_ref):
    g = dinv_ref[...] * (agg_ref[...] + hn_ref[...])
    mulv = jnp.dot(g, w_ref[...], preferred_element_type=jnp.float32) \
        + b_ref[...]
    l = mu_ref.shape[1]
    mu = mulv[:, :l]
    lv = mulv[:, l:]
    mu_ref[...] = mu
    lv_ref[...] = lv
    z_ref[...] = mu + eps_ref[...] * jnp.exp(0.5 * lv)


def _head(agg, hn, dinv_col, wml, bml, eps, block_rows=1000):
    n, l = eps.shape
    d = hn.shape[1]
    h2 = wml.shape[1]
    grid = (n // block_rows,)
    out_shape = [jax.ShapeDtypeStruct((n, l), jnp.float32)] * 3
    return pl.pallas_call(
        _head_body,
        grid=grid,
        in_specs=[
            pl.BlockSpec((block_rows, d), lambda i: (i, 0)),
            pl.BlockSpec((block_rows, d), lambda i: (i, 0)),
            pl.BlockSpec((block_rows, 1), lambda i: (i, 0)),
            pl.BlockSpec((d, h2), lambda i: (0, 0)),
            pl.BlockSpec((1, h2), lambda i: (0, 0)),
            pl.BlockSpec((block_rows, l), lambda i: (i, 0)),
        ],
        out_specs=[pl.BlockSpec((block_rows, l), lambda i: (i, 0))] * 3,
        out_shape=out_shape,
    )(agg, hn, dinv_col, wml, bml, eps)


def _decode_body(z_ref, zt_ref, o_ref):
    acc = jnp.dot(z_ref[...], zt_ref[...], preferred_element_type=jnp.float32)
    o_ref[...] = jax.nn.sigmoid(acc)


def _decode(z, block_rows=512, block_cols=2048):
    n, l = z.shape
    zt = z.T
    grid = (pl.cdiv(n, block_rows), pl.cdiv(n, block_cols))
    return pl.pallas_call(
        _decode_body,
        grid=grid,
        in_specs=[
            pl.BlockSpec((block_rows, l), lambda i, j: (i, 0)),
            pl.BlockSpec((l, block_cols), lambda i, j: (0, j)),
        ],
        out_specs=pl.BlockSpec((block_rows, block_cols), lambda i, j: (i, j)),
        out_shape=jax.ShapeDtypeStruct((n, n), jnp.float32),
    )(z, zt)


def kernel(x, edge_index, W1, b1, Wmu, bmu, Wlv, blv, eps):
    n = x.shape[0]
    e = edge_index.shape[1]

    grp = NSUB * CHUNK * NBUF
    ept = ((e + grp - 1) // grp) * CHUNK * NBUF
    e_pad = ept * NSUB
    n_spare = N_PAD - n

    src = edge_index[0]
    dst = edge_index[1]
    pad_idx = (n + jnp.arange(e_pad - e, dtype=jnp.int32) % n_spare)
    src3 = jnp.concatenate([src, pad_idx]).reshape(NSUB, ept // CHUNK, CHUNK)
    dst3 = jnp.concatenate([dst, pad_idx]).reshape(NSUB, ept // CHUNK, CHUNK)

    xw = _mm(x, W1, N_PAD)
    deg_t = _sc_deg_call(dst3)
    deg = 1.0 + deg_t[:, :HALF, 0].reshape(N_PAD)
    dinv_col = jax.lax.rsqrt(deg)[:, None]

    hn1 = _scale(xw, dinv_col)
    agg1 = _sc_agg_call(hn1, src3, dst3)[:, :HALF].reshape(N_PAD, -1)

    hn2 = _layer1(agg1, hn1, dinv_col, b1[None, :])
    agg2 = _sc_agg_call(hn2, src3, dst3)[:, :HALF].reshape(N_PAD, -1)

    wml = jnp.concatenate([Wmu, Wlv], axis=1)
    bml = jnp.concatenate([bmu, blv])[None, :]
    mu, logvar, z = _head(agg2[:n], hn2[:n], dinv_col[:n], wml, bml, eps)
    recon = _decode(z)
    return (recon, mu, logvar)


# speedup vs baseline: 17.3844x; 1.0428x over previous
import functools

import jax
import jax.numpy as jnp
from jax import lax
from jax.experimental import pallas as pl
from jax.experimental.pallas import tpu as pltpu
from jax.experimental.pallas import tpu_sc as plsc

N_PAD = 10240
HALF = N_PAD // 2
TRASH = 1024
R = HALF + TRASH
CHUNK = 128
NBUF = 2
NSUB = 16


def _remap(dst_v, j, chalf):
    for g in range(CHUNK // 16):
        v = dst_v[j, pl.ds(g * 16, 16)]
        rel = v - chalf
        inb = (rel >= 0) & (rel < HALF)
        tr = HALF + (v & (TRASH - 1))
        dst_v[j, pl.ds(g * 16, 16)] = jnp.where(inb, rel, tr)


def _sc_agg_call(hn_pad, src3, dst3):
    n_pad, w = hn_pad.shape
    n_chunks = src3.shape[1]
    rpt = R // NSUB
    nzc = rpt // CHUNK
    assert rpt % CHUNK == 0 and n_chunks % NBUF == 0
    mesh = plsc.VectorSubcoreMesh(core_axis_name="c", subcore_axis_name="s")

    @functools.partial(
        pl.kernel,
        out_type=pltpu.HBM((2, R, w), jnp.float32),
        mesh=mesh,
        scratch_types=[
            pltpu.VMEM((n_chunks, CHUNK), jnp.int32),
            pltpu.VMEM((n_chunks, CHUNK), jnp.int32),
            *[pltpu.VMEM((CHUNK, w), jnp.float32) for _ in range(NBUF)],
            pltpu.VMEM_SHARED((R, w), jnp.float32),
            *[pltpu.SemaphoreType.DMA for _ in range(NBUF)],
        ],
    )
    def k(hn_hbm, src_hbm, dst_hbm, zpat_hbm, out_hbm, src_v, dst_v, *rest):
        bufs = rest[:NBUF]
        table = rest[NBUF]
        sems = rest[NBUF + 1:]
        c = lax.axis_index("c")
        s = lax.axis_index("s")
        r0 = s * rpt
        chalf = c * HALF

        pltpu.sync_copy(zpat_hbm, bufs[0])
        for q in range(nzc):
            pltpu.sync_copy(bufs[0], table.at[pl.ds(r0 + q * CHUNK, CHUNK)])

        pltpu.sync_copy(src_hbm.at[s], src_v)
        pltpu.sync_copy(dst_hbm.at[s], dst_v)
        plsc.subcore_barrier()

        for b in range(NBUF):
            pltpu.async_copy(hn_hbm.at[src_v.at[b]], bufs[b], sems[b])

        def group(t, carry):
            base = t * NBUF
            for b in range(NBUF):
                _remap(dst_v, base + b, chalf)
                pltpu.make_async_copy(
                    hn_hbm.at[src_v.at[base + b]], bufs[b], sems[b]).wait()
                pltpu.sync_copy(bufs[b], table.at[dst_v.at[base + b]],
                                add=True)
                pltpu.async_copy(
                    hn_hbm.at[src_v.at[base + NBUF + b]], bufs[b], sems[b])
            return carry
        lax.fori_loop(0, n_chunks // NBUF - 1, group, 0)

        base = n_chunks - NBUF
        for b in range(NBUF):
            _remap(dst_v, base + b, chalf)
            pltpu.make_async_copy(
                hn_hbm.at[src_v.at[base + b]], bufs[b], sems[b]).wait()
            pltpu.sync_copy(bufs[b], table.at[dst_v.at[base + b]], add=True)
        plsc.subcore_barrier()

        pltpu.sync_copy(
            table.at[pl.ds(r0, rpt)],
            out_hbm.at[c, pl.ds(r0, rpt)],
        )

    zpat = jnp.zeros((CHUNK, w), jnp.float32)
    return k(hn_pad, src3, dst3, zpat)


def _sc_deg_call(dst3):
    w = 16
    n_chunks = dst3.shape[1]
    rpt = R // NSUB
    nzc = rpt // CHUNK
    mesh = plsc.VectorSubcoreMesh(core_axis_name="c", subcore_axis_name="s")

    @functools.partial(
        pl.kernel,
        out_type=pltpu.HBM((2, R, w), jnp.float32),
        mesh=mesh,
        scratch_types=[
            pltpu.VMEM((n_chunks, CHUNK), jnp.int32),
            pltpu.VMEM((CHUNK, w), jnp.float32),
            pltpu.VMEM((CHUNK, w), jnp.float32),
            pltpu.VMEM_SHARED((R, w), jnp.float32),
        ],
    )
    def k(dst_hbm, zpat_hbm, opat_hbm, out_hbm, dst_v, zbuf, obuf, table):
        c = lax.axis_index("c")
        s = lax.axis_index("s")
        r0 = s * rpt
        chalf = c * HALF

        pltpu.sync_copy(zpat_hbm, zbuf)
        pltpu.sync_copy(opat_hbm, obuf)
        for q in range(nzc):
            pltpu.sync_copy(zbuf, table.at[pl.ds(r0 + q * CHUNK, CHUNK)])
        pltpu.sync_copy(dst_hbm.at[s], dst_v)
        plsc.subcore_barrier()

        def chunk(j, carry):
            _remap(dst_v, j, chalf)
            pltpu.sync_copy(obuf, table.at[dst_v.at[j]], add=True)
            return carry
        lax.fori_loop(0, n_chunks, chunk, 0)
        plsc.subcore_barrier()

        pltpu.sync_copy(
            table.at[pl.ds(r0, rpt)],
            out_hbm.at[c, pl.ds(r0, rpt)],
        )

    zpat = jnp.zeros((CHUNK, w), jnp.float32)
    opat = jnp.tile(jax.nn.one_hot(0, w, dtype=jnp.float32), (CHUNK, 1))
    return k(dst3, zpat, opat)


def _mm_body(x_ref, w_ref, o_ref):
    o_ref[...] = jnp.dot(x_ref[...], w_ref[...],
                         preferred_element_type=jnp.float32)


def _mm(x, w, n_out, block_rows=1024):
    n, d = x.shape
    h = w.shape[1]
    grid = (n_out // block_rows,)
    return pl.pallas_call(
        _mm_body,
        grid=grid,
        in_specs=[
            pl.BlockSpec((block_rows, d), lambda i: (i, 0)),
            pl.BlockSpec((d, h), lambda i: (0, 0)),
        ],
        out_specs=pl.BlockSpec((block_rows, h), lambda i: (i, 0)),
        out_shape=jax.ShapeDtypeStruct((n_out, h), jnp.float32),
    )(x, w)


def _scale_body(x_ref, dinv_ref, o_ref):
    o_ref[...] = x_ref[...] * dinv_ref[...]


def _scale(x, dinv_col, block_rows=1024):
    n, d = x.shape
    grid = (n // block_rows,)
    return pl.pallas_call(
        _scale_body,
        grid=grid,
        in_specs=[
            pl.BlockSpec((block_rows, d), lambda i: (i, 0)),
            pl.BlockSpec((block_rows, 1), lambda i: (i, 0)),
        ],
        out_specs=pl.BlockSpec((block_rows, d), lambda i: (i, 0)),
        out_shape=jax.ShapeDtypeStruct((n, d), jnp.float32),
    )(x, dinv_col)


def _layer1_body(agg_ref, hn_ref, dinv_ref, b_ref, o_ref):
    h = jnp.maximum(
        dinv_ref[...] * (agg_ref[...] + hn_ref[...]) + b_ref[...], 0.0)
    o_ref[...] = h * dinv_ref[...]


def _layer1(agg, hn, dinv_col, b_row, block_rows=1024):
    n, d = hn.shape
    grid = (n // block_rows,)
    return pl.pallas_call(
        _layer1_body,
        grid=grid,
        in_specs=[
            pl.BlockSpec((block_rows, d), lambda i: (i, 0)),
            pl.BlockSpec((block_rows, d), lambda i: (i, 0)),
            pl.BlockSpec((block_rows, 1), lambda i: (i, 0)),
            pl.BlockSpec((1, d), lambda i: (0, 0)),
        ],
        out_specs=pl.BlockSpec((block_rows, d), lambda i: (i, 0)),
        out_shape=jax.ShapeDtypeStruct((n, d), jnp.float32),
    )(agg, hn, dinv_col, b_row)


def _head_body(agg_ref, hn_ref, dinv_ref, w_ref, b_ref, eps_ref,
               mu_ref, lv_ref, z_ref):
    g = dinv_ref[...] * (agg_ref[...] + hn_ref[...])
    mulv = jnp.dot(g, w_ref[...], preferred_element_type=jnp.float32) \
        + b_ref[...]
    l = mu_ref.shape[1]
    mu = mulv[:, :l]
    lv = mulv[:, l:]
    mu_ref[...] = mu
    lv_ref[...] = lv
    z_ref[...] = mu + eps_ref[...] * jnp.exp(0.5 * lv)


def _head(agg, hn, dinv_col, wml, bml, eps, block_rows=1000):
    n, l = eps.shape
    d = hn.shape[1]
    h2 = wml.shape[1]
    grid = (n // block_rows,)
    out_shape = [jax.ShapeDtypeStruct((n, l), jnp.float32)] * 3
    return pl.pallas_call(
        _head_body,
        grid=grid,
        in_specs=[
            pl.BlockSpec((block_rows, d), lambda i: (i, 0)),
            pl.BlockSpec((block_rows, d), lambda i: (i, 0)),
            pl.BlockSpec((block_rows, 1), lambda i: (i, 0)),
            pl.BlockSpec((d, h2), lambda i: (0, 0)),
            pl.BlockSpec((1, h2), lambda i: (0, 0)),
            pl.BlockSpec((block_rows, l), lambda i: (i, 0)),
        ],
        out_specs=[pl.BlockSpec((block_rows, l), lambda i: (i, 0))] * 3,
        out_shape=out_shape,
    )(agg, hn, dinv_col, wml, bml, eps)


def _decode_body(z_ref, zt_ref, o_ref):
    acc = jnp.dot(z_ref[...], zt_ref[...], preferred_element_type=jnp.float32)
    o_ref[...] = jax.nn.sigmoid(acc)


def _decode(z, block_rows=1024, block_cols=2048):
    n, l = z.shape
    zt = z.T
    grid = (pl.cdiv(n, block_rows), pl.cdiv(n, block_cols))
    return pl.pallas_call(
        _decode_body,
        grid=grid,
        in_specs=[
            pl.BlockSpec((block_rows, l), lambda i, j: (i, 0)),
            pl.BlockSpec((l, block_cols), lambda i, j: (0, j)),
        ],
        out_specs=pl.BlockSpec((block_rows, block_cols), lambda i, j: (i, j)),
        out_shape=jax.ShapeDtypeStruct((n, n), jnp.float32),
    )(z, zt)


def kernel(x, edge_index, W1, b1, Wmu, bmu, Wlv, blv, eps):
    n = x.shape[0]
    e = edge_index.shape[1]

    grp = NSUB * CHUNK * NBUF
    ept = ((e + grp - 1) // grp) * CHUNK * NBUF
    e_pad = ept * NSUB
    n_spare = N_PAD - n

    src = edge_index[0]
    dst = edge_index[1]
    pad_idx = (n + jnp.arange(e_pad - e, dtype=jnp.int32) % n_spare)
    src3 = jnp.concatenate([src, pad_idx]).reshape(NSUB, ept // CHUNK, CHUNK)
    dst3 = jnp.concatenate([dst, pad_idx]).reshape(NSUB, ept // CHUNK, CHUNK)

    xw = _mm(x, W1, N_PAD)
    deg_t = _sc_deg_call(dst3)
    deg = 1.0 + deg_t[:, :HALF, 0].reshape(N_PAD)
    dinv_col = jax.lax.rsqrt(deg)[:, None]

    hn1 = _scale(xw, dinv_col)
    agg1 = _sc_agg_call(hn1, src3, dst3)[:, :HALF].reshape(N_PAD, -1)

    hn2 = _layer1(agg1, hn1, dinv_col, b1[None, :])
    agg2 = _sc_agg_call(hn2, src3, dst3)[:, :HALF].reshape(N_PAD, -1)

    wml = jnp.concatenate([Wmu, Wlv], axis=1)
    bml = jnp.concatenate([bmu, blv])[None, :]
    mu, logvar, z = _head(agg2[:n], hn2[:n], dinv_col[:n], wml, bml, eps)
    recon = _decode(z)
    return (recon, mu, logvar)


# speedup vs baseline: 17.6278x; 1.0140x over previous
import functools

import jax
import jax.numpy as jnp
from jax import lax
from jax.experimental import pallas as pl
from jax.experimental.pallas import tpu as pltpu
from jax.experimental.pallas import tpu_sc as plsc

N_PAD = 10240
HALF = N_PAD // 2
TRASH = 1024
R = HALF + TRASH
CHUNK = 128
NBUF = 2
NSUB = 16


def _remap(dst_v, j, chalf):
    for g in range(CHUNK // 16):
        v = dst_v[j, pl.ds(g * 16, 16)]
        rel = v - chalf
        inb = (rel >= 0) & (rel < HALF)
        tr = HALF + (v & (TRASH - 1))
        dst_v[j, pl.ds(g * 16, 16)] = jnp.where(inb, rel, tr)


def _sc_agg_call(hn_pad, src3, dst3):
    n_pad, w = hn_pad.shape
    n_chunks = src3.shape[1]
    rpt = R // NSUB
    nzc = rpt // CHUNK
    assert rpt % CHUNK == 0 and n_chunks % NBUF == 0
    mesh = plsc.VectorSubcoreMesh(core_axis_name="c", subcore_axis_name="s")

    @functools.partial(
        pl.kernel,
        out_type=pltpu.HBM((2, R, w), jnp.float32),
        mesh=mesh,
        scratch_types=[
            pltpu.VMEM((n_chunks, CHUNK), jnp.int32),
            pltpu.VMEM((n_chunks, CHUNK), jnp.int32),
            *[pltpu.VMEM((CHUNK, w), jnp.float32) for _ in range(NBUF)],
            pltpu.VMEM_SHARED((R, w), jnp.float32),
            *[pltpu.SemaphoreType.DMA for _ in range(NBUF)],
        ],
    )
    def k(hn_hbm, src_hbm, dst_hbm, zpat_hbm, out_hbm, src_v, dst_v, *rest):
        bufs = rest[:NBUF]
        table = rest[NBUF]
        sems = rest[NBUF + 1:]
        c = lax.axis_index("c")
        s = lax.axis_index("s")
        r0 = s * rpt
        chalf = c * HALF

        pltpu.sync_copy(zpat_hbm, bufs[0])
        for q in range(nzc):
            pltpu.sync_copy(bufs[0], table.at[pl.ds(r0 + q * CHUNK, CHUNK)])

        pltpu.sync_copy(src_hbm.at[s], src_v)
        pltpu.sync_copy(dst_hbm.at[s], dst_v)
        plsc.subcore_barrier()

        for b in range(NBUF):
            pltpu.async_copy(hn_hbm.at[src_v.at[b]], bufs[b], sems[b])

        def group(t, carry):
            base = t * NBUF
            for b in range(NBUF):
                _remap(dst_v, base + b, chalf)
                pltpu.make_async_copy(
                    hn_hbm.at[src_v.at[base + b]], bufs[b], sems[b]).wait()
                pltpu.sync_copy(bufs[b], table.at[dst_v.at[base + b]],
                                add=True)
                pltpu.async_copy(
                    hn_hbm.at[src_v.at[base + NBUF + b]], bufs[b], sems[b])
            return carry
        lax.fori_loop(0, n_chunks // NBUF - 1, group, 0)

        base = n_chunks - NBUF
        for b in range(NBUF):
            _remap(dst_v, base + b, chalf)
            pltpu.make_async_copy(
                hn_hbm.at[src_v.at[base + b]], bufs[b], sems[b]).wait()
            pltpu.sync_copy(bufs[b], table.at[dst_v.at[base + b]], add=True)
        plsc.subcore_barrier()

        pltpu.sync_copy(
            table.at[pl.ds(r0, rpt)],
            out_hbm.at[c, pl.ds(r0, rpt)],
        )

    zpat = jnp.zeros((CHUNK, w), jnp.float32)
    return k(hn_pad, src3, dst3, zpat)


def _sc_deg_call(dst3):
    w = 16
    n_chunks = dst3.shape[1]
    rpt = R // NSUB
    nzc = rpt // CHUNK
    mesh = plsc.VectorSubcoreMesh(core_axis_name="c", subcore_axis_name="s")

    @functools.partial(
        pl.kernel,
        out_type=pltpu.HBM((2, R, w), jnp.float32),
        mesh=mesh,
        scratch_types=[
            pltpu.VMEM((n_chunks, CHUNK), jnp.int32),
            pltpu.VMEM((CHUNK, w), jnp.float32),
            pltpu.VMEM((CHUNK, w), jnp.float32),
            pltpu.VMEM_SHARED((R, w), jnp.float32),
        ],
    )
    def k(dst_hbm, zpat_hbm, opat_hbm, out_hbm, dst_v, zbuf, obuf, table):
        c = lax.axis_index("c")
        s = lax.axis_index("s")
        r0 = s * rpt
        chalf = c * HALF

        pltpu.sync_copy(zpat_hbm, zbuf)
        pltpu.sync_copy(opat_hbm, obuf)
        for q in range(nzc):
            pltpu.sync_copy(zbuf, table.at[pl.ds(r0 + q * CHUNK, CHUNK)])
        pltpu.sync_copy(dst_hbm.at[s], dst_v)
        plsc.subcore_barrier()

        def chunk(j, carry):
            _remap(dst_v, j, chalf)
            pltpu.sync_copy(obuf, table.at[dst_v.at[j]], add=True)
            return carry
        lax.fori_loop(0, n_chunks, chunk, 0)
        plsc.subcore_barrier()

        pltpu.sync_copy(
            table.at[pl.ds(r0, rpt)],
            out_hbm.at[c, pl.ds(r0, rpt)],
        )

    zpat = jnp.zeros((CHUNK, w), jnp.float32)
    opat = jnp.tile(jax.nn.one_hot(0, w, dtype=jnp.float32), (CHUNK, 1))
    return k(dst3, zpat, opat)


def _mm_body(x_ref, w_ref, o_ref):
    o_ref[...] = jnp.dot(x_ref[...], w_ref[...],
                         preferred_element_type=jnp.float32)


def _mm(x, w, n_out, block_rows=1024):
    n, d = x.shape
    h = w.shape[1]
    grid = (n_out // block_rows,)
    return pl.pallas_call(
        _mm_body,
        grid=grid,
        in_specs=[
            pl.BlockSpec((block_rows, d), lambda i: (i, 0)),
            pl.BlockSpec((d, h), lambda i: (0, 0)),
        ],
        out_specs=pl.BlockSpec((block_rows, h), lambda i: (i, 0)),
        out_shape=jax.ShapeDtypeStruct((n_out, h), jnp.float32),
    )(x, w)


def _scale_body(x_ref, dinv_ref, o_ref):
    o_ref[...] = x_ref[...] * dinv_ref[...]


def _scale(x, dinv_col, block_rows=1024):
    n, d = x.shape
    grid = (n // block_rows,)
    return pl.pallas_call(
        _scale_body,
        grid=grid,
        in_specs=[
            pl.BlockSpec((block_rows, d), lambda i: (i, 0)),
            pl.BlockSpec((block_rows, 1), lambda i: (i, 0)),
        ],
        out_specs=pl.BlockSpec((block_rows, d), lambda i: (i, 0)),
        out_shape=jax.ShapeDtypeStruct((n, d), jnp.float32),
    )(x, dinv_col)


def _layer1_body(agg_ref, hn_ref, dinv_ref, b_ref, o_ref):
    h = jnp.maximum(
        dinv_ref[...] * (agg_ref[...] + hn_ref[...]) + b_ref[...], 0.0)
    o_ref[...] = h * dinv_ref[...]


def _layer1(agg, hn, dinv_col, b_row, block_rows=1024):
    n, d = hn.shape
    grid = (n // block_rows,)
    return pl.pallas_call(
        _layer1_body,
        grid=grid,
        in_specs=[
            pl.BlockSpec((block_rows, d), lambda i: (i, 0)),
            pl.BlockSpec((block_rows, d), lambda i: (i, 0)),
            pl.BlockSpec((block_rows, 1), lambda i: (i, 0)),
            pl.BlockSpec((1, d), lambda i: (0, 0)),
        ],
        out_specs=pl.BlockSpec((block_rows, d), lambda i: (i, 0)),
        out_shape=jax.ShapeDtypeStruct((n, d), jnp.float32),
    )(agg, hn, dinv_col, b_row)


def _head_body(agg_ref, hn_ref, dinv_ref, w_ref, b_ref, eps_ref,
               mu_ref, lv_ref, z_ref):
    g = dinv_ref[...] * (agg_ref[...] + hn_ref[...])
    mulv = jnp.dot(g, w_ref[...], preferred_element_type=jnp.float32) \
        + b_ref[...]
    l = mu_ref.shape[1]
    mu = mulv[:, :l]
    lv = mulv[:, l:]
    mu_ref[...] = mu
    lv_ref[...] = lv
    z_ref[...] = mu + eps_ref[...] * jnp.exp(0.5 * lv)


def _head(agg, hn, dinv_col, wml, bml, eps, block_rows=1000):
    n, l = eps.shape
    d = hn.shape[1]
    h2 = wml.shape[1]
    grid = (n // block_rows,)
    out_shape = [jax.ShapeDtypeStruct((n, l), jnp.float32)] * 3
    return pl.pallas_call(
        _head_body,
        grid=grid,
        in_specs=[
            pl.BlockSpec((block_rows, d), lambda i: (i, 0)),
            pl.BlockSpec((block_rows, d), lambda i: (i, 0)),
            pl.BlockSpec((block_rows, 1), lambda i: (i, 0)),
            pl.BlockSpec((d, h2), lambda i: (0, 0)),
            pl.BlockSpec((1, h2), lambda i: (0, 0)),
            pl.BlockSpec((block_rows, l), lambda i: (i, 0)),
        ],
        out_specs=[pl.BlockSpec((block_rows, l), lambda i: (i, 0))] * 3,
        out_shape=out_shape,
    )(agg, hn, dinv_col, wml, bml, eps)


def _decode_body(z_ref, zt_ref, o_ref):
    acc = jnp.dot(z_ref[...], zt_ref[...], preferred_element_type=jnp.float32)
    o_ref[...] = jax.nn.sigmoid(acc)


def _decode(z, block_rows=2048, block_cols=2048):
    n, l = z.shape
    zt = z.T
    grid = (pl.cdiv(n, block_rows), pl.cdiv(n, block_cols))
    return pl.pallas_call(
        _decode_body,
        grid=grid,
        in_specs=[
            pl.BlockSpec((block_rows, l), lambda i, j: (i, 0)),
            pl.BlockSpec((l, block_cols), lambda i, j: (0, j)),
        ],
        out_specs=pl.BlockSpec((block_rows, block_cols), lambda i, j: (i, j)),
        out_shape=jax.ShapeDtypeStruct((n, n), jnp.float32),
    )(z, zt)


def kernel(x, edge_index, W1, b1, Wmu, bmu, Wlv, blv, eps):
    n = x.shape[0]
    e = edge_index.shape[1]

    grp = NSUB * CHUNK * NBUF
    ept = ((e + grp - 1) // grp) * CHUNK * NBUF
    e_pad = ept * NSUB
    n_spare = N_PAD - n

    src = edge_index[0]
    dst = edge_index[1]
    pad_idx = (n + jnp.arange(e_pad - e, dtype=jnp.int32) % n_spare)
    src3 = jnp.concatenate([src, pad_idx]).reshape(NSUB, ept // CHUNK, CHUNK)
    dst3 = jnp.concatenate([dst, pad_idx]).reshape(NSUB, ept // CHUNK, CHUNK)

    xw = _mm(x, W1, N_PAD)
    deg_t = _sc_deg_call(dst3)
    deg = 1.0 + deg_t[:, :HALF, 0].reshape(N_PAD)
    dinv_col = jax.lax.rsqrt(deg)[:, None]

    hn1 = _scale(xw, dinv_col)
    agg1 = _sc_agg_call(hn1, src3, dst3)[:, :HALF].reshape(N_PAD, -1)

    hn2 = _layer1(agg1, hn1, dinv_col, b1[None, :])
    agg2 = _sc_agg_call(hn2, src3, dst3)[:, :HALF].reshape(N_PAD, -1)

    wml = jnp.concatenate([Wmu, Wlv], axis=1)
    bml = jnp.concatenate([bmu, blv])[None, :]
    mu, logvar, z = _head(agg2[:n], hn2[:n], dinv_col[:n], wml, bml, eps)
    recon = _decode(z)
    return (recon, mu, logvar)


# speedup vs baseline: 18.7017x; 1.0609x over previous
import functools

import jax
import jax.numpy as jnp
from jax import lax
from jax.experimental import pallas as pl
from jax.experimental.pallas import tpu as pltpu
from jax.experimental.pallas import tpu_sc as plsc

N_PAD = 10240
HALF = N_PAD // 2
TRASH = 1024
R = HALF + TRASH
CHUNK = 128
NBUF = 3
NSUB = 16


def _remap(dst_v, j, chalf):
    for g in range(CHUNK // 16):
        v = dst_v[j, pl.ds(g * 16, 16)]
        rel = v - chalf
        inb = (rel >= 0) & (rel < HALF)
        tr = HALF + (v & (TRASH - 1))
        dst_v[j, pl.ds(g * 16, 16)] = jnp.where(inb, rel, tr)


def _sc_agg_call(hn_pad, src_a, src_b, dst_a, dst_b):
    n_pad, w = hn_pad.shape
    hc = src_a.shape[1]
    rpt = R // NSUB
    nzc = rpt // CHUNK
    assert rpt % CHUNK == 0 and hc % 8 == 0
    mesh = plsc.VectorSubcoreMesh(core_axis_name="c", subcore_axis_name="s")

    @functools.partial(
        pl.kernel,
        out_type=pltpu.HBM((2, R, w), jnp.float32),
        mesh=mesh,
        scratch_types=[
            pltpu.VMEM((hc, CHUNK), jnp.int32),
            pltpu.VMEM((hc, CHUNK), jnp.int32),
            *[pltpu.VMEM((CHUNK, w), jnp.float32) for _ in range(NBUF)],
            pltpu.VMEM_SHARED((R, w), jnp.float32),
            *[pltpu.SemaphoreType.DMA for _ in range(NBUF)],
        ],
    )
    def k(hn_hbm, sa_hbm, sb_hbm, da_hbm, db_hbm, zpat_hbm, out_hbm,
          src_v, dst_v, *rest):
        bufs = rest[:NBUF]
        table = rest[NBUF]
        sems = rest[NBUF + 1:]
        c = lax.axis_index("c")
        s = lax.axis_index("s")
        r0 = s * rpt
        chalf = c * HALF

        pltpu.sync_copy(zpat_hbm, bufs[0])
        for q in range(nzc):
            pltpu.sync_copy(bufs[0], table.at[pl.ds(r0 + q * CHUNK, CHUNK)])
        plsc.subcore_barrier()

        for s_hbm, d_hbm in ((sa_hbm, da_hbm), (sb_hbm, db_hbm)):
            pltpu.sync_copy(s_hbm.at[s], src_v)
            pltpu.sync_copy(d_hbm.at[s], dst_v)

            for b in range(NBUF):
                pltpu.async_copy(hn_hbm.at[src_v.at[b]], bufs[b], sems[b])

            def group(t, carry):
                base = t * NBUF
                for b in range(NBUF):
                    _remap(dst_v, base + b, chalf)
                    pltpu.make_async_copy(
                        hn_hbm.at[src_v.at[base + b]], bufs[b],
                        sems[b]).wait()
                    pltpu.sync_copy(bufs[b], table.at[dst_v.at[base + b]],
                                    add=True)
                    pltpu.async_copy(
                        hn_hbm.at[src_v.at[base + NBUF + b]], bufs[b],
                        sems[b])
                return carry
            g_full = hc // NBUF - 1
            lax.fori_loop(0, g_full, group, 0)

            for j in range(g_full * NBUF, hc):
                b = j % NBUF
                _remap(dst_v, j, chalf)
                pltpu.make_async_copy(
                    hn_hbm.at[src_v.at[j]], bufs[b], sems[b]).wait()
                pltpu.sync_copy(bufs[b], table.at[dst_v.at[j]], add=True)
                if j + NBUF < hc:
                    pltpu.async_copy(
                        hn_hbm.at[src_v.at[j + NBUF]], bufs[b], sems[b])
        plsc.subcore_barrier()

        pltpu.sync_copy(
            table.at[pl.ds(r0, rpt)],
            out_hbm.at[c, pl.ds(r0, rpt)],
        )

    zpat = jnp.zeros((CHUNK, w), jnp.float32)
    return k(hn_pad, src_a, src_b, dst_a, dst_b, zpat)


def _sc_deg_call(dst3):
    w = 16
    n_chunks = dst3.shape[1]
    rpt = R // NSUB
    nzc = rpt // CHUNK
    mesh = plsc.VectorSubcoreMesh(core_axis_name="c", subcore_axis_name="s")

    @functools.partial(
        pl.kernel,
        out_type=pltpu.HBM((2, R, w), jnp.float32),
        mesh=mesh,
        scratch_types=[
            pltpu.VMEM((n_chunks, CHUNK), jnp.int32),
            pltpu.VMEM((CHUNK, w), jnp.float32),
            pltpu.VMEM((CHUNK, w), jnp.float32),
            pltpu.VMEM_SHARED((R, w), jnp.float32),
        ],
    )
    def k(dst_hbm, zpat_hbm, opat_hbm, out_hbm, dst_v, zbuf, obuf, table):
        c = lax.axis_index("c")
        s = lax.axis_index("s")
        r0 = s * rpt
        chalf = c * HALF

        pltpu.sync_copy(zpat_hbm, zbuf)
        pltpu.sync_copy(opat_hbm, obuf)
        for q in range(nzc):
            pltpu.sync_copy(zbuf, table.at[pl.ds(r0 + q * CHUNK, CHUNK)])
        pltpu.sync_copy(dst_hbm.at[s], dst_v)
        plsc.subcore_barrier()

        def chunk(j, carry):
            _remap(dst_v, j, chalf)
            pltpu.sync_copy(obuf, table.at[dst_v.at[j]], add=True)
            return carry
        lax.fori_loop(0, n_chunks, chunk, 0)
        plsc.subcore_barrier()

        pltpu.sync_copy(
            table.at[pl.ds(r0, rpt)],
            out_hbm.at[c, pl.ds(r0, rpt)],
        )

    zpat = jnp.zeros((CHUNK, w), jnp.float32)
    opat = jnp.tile(jax.nn.one_hot(0, w, dtype=jnp.float32), (CHUNK, 1))
    return k(dst3, zpat, opat)


def _mm_body(x_ref, w_ref, o_ref):
    o_ref[...] = jnp.dot(x_ref[...], w_ref[...],
                         preferred_element_type=jnp.float32)


def _mm(x, w, n_out, block_rows=1024):
    n, d = x.shape
    h = w.shape[1]
    grid = (n_out // block_rows,)
    return pl.pallas_call(
        _mm_body,
        grid=grid,
        in_specs=[
            pl.BlockSpec((block_rows, d), lambda i: (i, 0)),
            pl.BlockSpec((d, h), lambda i: (0, 0)),
        ],
        out_specs=pl.BlockSpec((block_rows, h), lambda i: (i, 0)),
        out_shape=jax.ShapeDtypeStruct((n_out, h), jnp.float32),
    )(x, w)


def _scale_body(x_ref, dinv_ref, o_ref):
    o_ref[...] = x_ref[...] * dinv_ref[...]


def _scale(x, dinv_col, block_rows=1024):
    n, d = x.shape
    grid = (n // block_rows,)
    return pl.pallas_call(
        _scale_body,
        grid=grid,
        in_specs=[
            pl.BlockSpec((block_rows, d), lambda i: (i, 0)),
            pl.BlockSpec((block_rows, 1), lambda i: (i, 0)),
        ],
        out_specs=pl.BlockSpec((block_rows, d), lambda i: (i, 0)),
        out_shape=jax.ShapeDtypeStruct((n, d), jnp.float32),
    )(x, dinv_col)


def _layer1_body(agg_ref, hn_ref, dinv_ref, b_ref, o_ref):
    h = jnp.maximum(
        dinv_ref[...] * (agg_ref[...] + hn_ref[...]) + b_ref[...], 0.0)
    o_ref[...] = h * dinv_ref[...]


def _layer1(agg, hn, dinv_col, b_row, block_rows=1024):
    n, d = hn.shape
    grid = (n // block_rows,)
    return pl.pallas_call(
        _layer1_body,
        grid=grid,
        in_specs=[
            pl.BlockSpec((block_rows, d), lambda i: (i, 0)),
            pl.BlockSpec((block_rows, d), lambda i: (i, 0)),
            pl.BlockSpec((block_rows, 1), lambda i: (i, 0)),
            pl.BlockSpec((1, d), lambda i: (0, 0)),
        ],
        out_specs=pl.BlockSpec((block_rows, d), lambda i: (i, 0)),
        out_shape=jax.ShapeDtypeStruct((n, d), jnp.float32),
    )(agg, hn, dinv_col, b_row)


def _head_body(agg_ref, hn_ref, dinv_ref, w_ref, b_ref, eps_ref,
               mu_ref, lv_ref, z_ref):
    g = dinv_ref[...] * (agg_ref[...] + hn_ref[...])
    mulv = jnp.dot(g, w_ref[...], preferred_element_type=jnp.float32) \
        + b_ref[...]
    l = mu_ref.shape[1]
    mu = mulv[:, :l]
    lv = mulv[:, l:]
    mu_ref[...] = mu
    lv_ref[...] = lv
    z_ref[...] = mu + eps_ref[...] * jnp.exp(0.5 * lv)


def _head(agg, hn, dinv_col, wml, bml, eps, block_rows=1000):
    n, l = eps.shape
    d = hn.shape[1]
    h2 = wml.shape[1]
    grid = (n // block_rows,)
    out_shape = [jax.ShapeDtypeStruct((n, l), jnp.float32)] * 3
    return pl.pallas_call(
        _head_body,
        grid=grid,
        in_specs=[
            pl.BlockSpec((block_rows, d), lambda i: (i, 0)),
            pl.BlockSpec((block_rows, d), lambda i: (i, 0)),
            pl.BlockSpec((block_rows, 1), lambda i: (i, 0)),
            pl.BlockSpec((d, h2), lambda i: (0, 0)),
            pl.BlockSpec((1, h2), lambda i: (0, 0)),
            pl.BlockSpec((block_rows, l), lambda i: (i, 0)),
        ],
        out_specs=[pl.BlockSpec((block_rows, l), lambda i: (i, 0))] * 3,
        out_shape=out_shape,
    )(agg, hn, dinv_col, wml, bml, eps)


def _decode_body(z_ref, zt_ref, o_ref):
    acc = jnp.dot(z_ref[...], zt_ref[...], preferred_element_type=jnp.float32)
    o_ref[...] = jax.nn.sigmoid(acc)


def _decode(z, block_rows=2048, block_cols=2048):
    n, l = z.shape
    zt = z.T
    grid = (pl.cdiv(n, block_rows), pl.cdiv(n, block_cols))
    return pl.pallas_call(
        _decode_body,
        grid=grid,
        in_specs=[
            pl.BlockSpec((block_rows, l), lambda i, j: (i, 0)),
            pl.BlockSpec((l, block_cols), lambda i, j: (0, j)),
        ],
        out_specs=pl.BlockSpec((block_rows, block_cols), lambda i, j: (i, j)),
        out_shape=jax.ShapeDtypeStruct((n, n), jnp.float32),
    )(z, zt)


def kernel(x, edge_index, W1, b1, Wmu, bmu, Wlv, blv, eps):
    n = x.shape[0]
    e = edge_index.shape[1]

    grp = NSUB * CHUNK * 16
    ept = ((e + grp - 1) // grp) * CHUNK * 16
    e_pad = ept * NSUB
    n_chunks = ept // CHUNK
    hc = n_chunks // 2
    n_spare = N_PAD - n

    src = edge_index[0]
    dst = edge_index[1]
    pad_idx = (n + jnp.arange(e_pad - e, dtype=jnp.int32) % n_spare)
    src3 = jnp.concatenate([src, pad_idx]).reshape(NSUB, n_chunks, CHUNK)
    dst3 = jnp.concatenate([dst, pad_idx]).reshape(NSUB, n_chunks, CHUNK)
    src_a, src_b = src3[:, :hc], src3[:, hc:]
    dst_a, dst_b = dst3[:, :hc], dst3[:, hc:]

    xw = _mm(x, W1, N_PAD)
    deg_t = _sc_deg_call(dst3)
    deg = 1.0 + deg_t[:, :HALF, 0].reshape(N_PAD)
    dinv_col = jax.lax.rsqrt(deg)[:, None]

    hn1 = _scale(xw, dinv_col)
    agg1 = _sc_agg_call(
        hn1, src_a, src_b, dst_a, dst_b)[:, :HALF].reshape(N_PAD, -1)

    hn2 = _layer1(agg1, hn1, dinv_col, b1[None, :])
    agg2 = _sc_agg_call(
        hn2, src_a, src_b, dst_a, dst_b)[:, :HALF].reshape(N_PAD, -1)

    wml = jnp.concatenate([Wmu, Wlv], axis=1)
    bml = jnp.concatenate([bmu, blv])[None, :]
    mu, logvar, z = _head(agg2[:n], hn2[:n], dinv_col[:n], wml, bml, eps)
    recon = _decode(z)
    return (recon, mu, logvar)
